# Initial kernel scaffold; baseline (speedup 1.0000x reference)
import jax
import jax.numpy as jnp
from jax.experimental import pallas as pl


def kernel(x, edge_index, W, b):
    raise NotImplementedError("write your pallas kernel here")


# speedup vs baseline: 7.0518x; 7.0518x over previous
import functools

import jax
import jax.numpy as jnp
from jax import lax
from jax.experimental import pallas as pl
from jax.experimental.pallas import tpu as pltpu
from jax.experimental.pallas import tpu_sc as plsc

_NUM_CORES = 2
_NUM_SUBCORES = 16
_NW = _NUM_CORES * _NUM_SUBCORES
_L = 16
_G = 8


def _mm_body(x_ref, w_ref, b_ref, tp_ref, tq_ref):
    c = w_ref.shape[0] // 2
    w1 = w_ref[:c, :]
    w2 = w_ref[c:, :]
    xb = x_ref[...]
    tp_ref[...] = jnp.dot(xb, w1 - w2, preferred_element_type=jnp.float32) + b_ref[...]
    tq_ref[...] = jnp.dot(xb, w2, preferred_element_type=jnp.float32)


def _project(x2, W, b2, n, c, out):
    blk = 2000
    grid = (n // blk,)
    return pl.pallas_call(
        _mm_body,
        grid=grid,
        in_specs=[
            pl.BlockSpec((blk, c), lambda i: (i, 0)),
            pl.BlockSpec((2 * c, out), lambda i: (0, 0)),
            pl.BlockSpec((1, out), lambda i: (0, 0)),
        ],
        out_specs=[
            pl.BlockSpec((blk, out), lambda i: (i, 0)),
            pl.BlockSpec((blk, out), lambda i: (i, 0)),
        ],
        out_shape=[
            jax.ShapeDtypeStruct((n, out), jnp.float32),
            jax.ShapeDtypeStruct((n, out), jnp.float32),
        ],
    )(x2, W, b2)


def _make_sc_kernel(npad, npw, k, out):
    mesh = plsc.VectorSubcoreMesh(core_axis_name="c", subcore_axis_name="s")
    nch = out // _L
    gk = _G * k
    ngrp = npw // _G
    nq = ngrp // 8

    @functools.partial(
        pl.kernel,
        out_type=jax.ShapeDtypeStruct((npad, out), jnp.float32),
        mesh=mesh,
        scratch_types=[
            pltpu.VMEM((nq, 8, gk), jnp.int32),
            pltpu.VMEM((nq, 8, gk), jnp.int32),
            pltpu.VMEM((gk, out), jnp.float32),
            pltpu.VMEM((gk, out), jnp.float32),
            pltpu.VMEM((_G, out), jnp.float32),
            pltpu.SemaphoreType.DMA,
            pltpu.SemaphoreType.DMA,
        ],
    )
    def sc_kernel(tp_hbm, tq_hbm, ei_hbm, ej_hbm, out_hbm,
                  ei_v, ej_v, bp, bq, ob, semp, semq):
        wid = lax.axis_index("s") * _NUM_CORES + lax.axis_index("c")
        base = wid * npw
        pltpu.sync_copy(ei_hbm.at[wid], ei_v)
        pltpu.sync_copy(ej_hbm.at[wid], ej_v)

        zero = jnp.zeros((_L,), jnp.float32)

        def body(q, carry):
            for r in range(8):
                cp = pltpu.async_copy(tp_hbm.at[ei_v.at[q, r]], bp, semp)
                cq = pltpu.async_copy(tq_hbm.at[ej_v.at[q, r]], bq, semq)
                cp.wait()
                cq.wait()

                def node(t, carry2):
                    for c in range(nch):
                        sl = pl.ds(c * _L, _L)
                        row = t * k
                        acc = bp[row, sl] + bq[row, sl]
                        for kk in range(1, k):
                            acc = jnp.maximum(acc, bp[row + kk, sl] + bq[row + kk, sl])
                        acc = jnp.maximum(acc, zero)
                        ob[t, sl] = acc
                    return carry2

                lax.fori_loop(0, _G, node, 0)
                pltpu.sync_copy(ob, out_hbm.at[pl.ds(base + (q * 8 + r) * _G, _G)])
            return carry

        lax.fori_loop(0, nq, body, 0)

    return sc_kernel


def kernel(x, edge_index, W, b):
    bsz, n, c = x.shape
    k = edge_index.shape[-1]
    out = W.shape[1]

    x2 = x.reshape(n, c)
    ei = edge_index[1].reshape(n, k)
    ej = edge_index[0].reshape(n, k)

    npw = -(-n // (64 * _NW)) * 64
    npad = npw * _NW
    if npad != n:
        pad = ((0, npad - n), (0, 0))
        ei = jnp.pad(ei, pad)
        ej = jnp.pad(ej, pad)

    gk = _G * k
    ngrp = npw // _G
    ei_g = ei.reshape(_NW, ngrp // 8, 8, gk)
    ej_g = ej.reshape(_NW, ngrp // 8, 8, gk)

    tp, tq = _project(x2, W, b.reshape(1, out), n, c, out)
    out_pad = _make_sc_kernel(npad, npw, k, out)(tp, tq, ei_g, ej_g)
    return out_pad[:n].reshape(bsz, n, out)


# speedup vs baseline: 8.8857x; 1.2601x over previous
import functools

import jax
import jax.numpy as jnp
from jax import lax
from jax.experimental import pallas as pl
from jax.experimental.pallas import tpu as pltpu
from jax.experimental.pallas import tpu_sc as plsc

_NUM_CORES = 2
_NUM_SUBCORES = 16
_NW = _NUM_CORES * _NUM_SUBCORES
_L = 16
_G = 4


def _mm_body(x_ref, w_ref, b_ref, tp_ref, tq_ref):
    c = w_ref.shape[0] // 2
    w1 = w_ref[:c, :]
    w2 = w_ref[c:, :]
    xb = x_ref[...]
    tp_ref[...] = jnp.dot(xb, w1 - w2, preferred_element_type=jnp.float32) + b_ref[...]
    tq_ref[...] = jnp.dot(xb, w2, preferred_element_type=jnp.float32)


def _project(x2, W, b2, n, c, out):
    blk = 2000
    grid = (n // blk,)
    return pl.pallas_call(
        _mm_body,
        grid=grid,
        in_specs=[
            pl.BlockSpec((blk, c), lambda i: (i, 0)),
            pl.BlockSpec((2 * c, out), lambda i: (0, 0)),
            pl.BlockSpec((1, out), lambda i: (0, 0)),
        ],
        out_specs=[
            pl.BlockSpec((blk, out), lambda i: (i, 0)),
            pl.BlockSpec((blk, out), lambda i: (i, 0)),
        ],
        out_shape=[
            jax.ShapeDtypeStruct((n, out), jnp.float32),
            jax.ShapeDtypeStruct((n, out), jnp.float32),
        ],
    )(x2, W, b2)


def _make_sc_kernel(npad, npw, k, out):
    mesh = plsc.VectorSubcoreMesh(core_axis_name="c", subcore_axis_name="s")
    nch = out // _L
    gk = _G * k
    ngrp = npw // _G
    npair = ngrp // 2
    nq = npair // 8

    @functools.partial(
        pl.kernel,
        out_type=jax.ShapeDtypeStruct((npad, out), jnp.float32),
        mesh=mesh,
        scratch_types=[
            pltpu.VMEM((nq, 8, 2 * gk), jnp.int32),
            pltpu.VMEM((nq, 8, 2 * gk), jnp.int32),
            pltpu.VMEM((2, gk, out), jnp.float32),
            pltpu.VMEM((2, gk, out), jnp.float32),
            pltpu.VMEM((2 * _G, out), jnp.float32),
            pltpu.SemaphoreType.DMA,
            pltpu.SemaphoreType.DMA,
            pltpu.SemaphoreType.DMA,
            pltpu.SemaphoreType.DMA,
        ],
    )
    def sc_kernel(tp_hbm, tq_hbm, ei_hbm, ej_hbm, out_hbm,
                  ei_v, ej_v, bp, bq, ob, semp0, semp1, semq0, semq1):
        wid = lax.axis_index("s") * _NUM_CORES + lax.axis_index("c")
        base = wid * npw
        pltpu.sync_copy(ei_hbm.at[wid], ei_v)
        pltpu.sync_copy(ej_hbm.at[wid], ej_v)

        semp = (semp0, semp1)
        semq = (semq0, semq1)
        zero = jnp.zeros((_L,), jnp.float32)

        def issue(qq, rr, r):
            sl = pl.ds(r * gk, gk)
            pltpu.async_copy(tp_hbm.at[ei_v.at[qq, rr, sl]], bp.at[r], semp[r])
            pltpu.async_copy(tq_hbm.at[ej_v.at[qq, rr, sl]], bq.at[r], semq[r])

        def drain(qq, rr, r):
            sl = pl.ds(r * gk, gk)
            pltpu.make_async_copy(tp_hbm.at[ei_v.at[qq, rr, sl]], bp.at[r], semp[r]).wait()
            pltpu.make_async_copy(tq_hbm.at[ej_v.at[qq, rr, sl]], bq.at[r], semq[r]).wait()

        issue(0, 0, 0)

        def body(gp, carry):
            qq = gp // 8
            rr = lax.rem(gp, 8)
            for r in range(2):
                if r == 0:
                    issue(qq, rr, 1)
                else:
                    @pl.when(gp + 1 < npair)
                    def _():
                        issue((gp + 1) // 8, lax.rem(gp + 1, 8), 0)
                drain(qq, rr, r)

                def node(t, carry2):
                    for c in range(nch):
                        sl = pl.ds(c * _L, _L)
                        row = t * k
                        acc = bp[r, row, sl] + bq[r, row, sl]
                        for kk in range(1, k):
                            acc = jnp.maximum(acc, bp[r, row + kk, sl] + bq[r, row + kk, sl])
                        acc = jnp.maximum(acc, zero)
                        ob[r * _G + t, sl] = acc
                    return carry2

                lax.fori_loop(0, _G, node, 0)
            pltpu.sync_copy(ob, out_hbm.at[pl.ds(base + gp * 2 * _G, 2 * _G)])
            return carry

        lax.fori_loop(0, npair, body, 0)

    return sc_kernel


def kernel(x, edge_index, W, b):
    bsz, n, c = x.shape
    k = edge_index.shape[-1]
    out = W.shape[1]

    x2 = x.reshape(n, c)
    ei = edge_index[1].reshape(n, k)
    ej = edge_index[0].reshape(n, k)

    npw = -(-n // (64 * _NW)) * 64
    npad = npw * _NW
    if npad != n:
        pad = ((0, npad - n), (0, 0))
        ei = jnp.pad(ei, pad)
        ej = jnp.pad(ej, pad)

    nq = npw * k // (8 * 2 * _G * k)
    ei_g = ei.reshape(_NW, nq, 8, 2 * _G * k)
    ej_g = ej.reshape(_NW, nq, 8, 2 * _G * k)

    tp, tq = _project(x2, W, b.reshape(1, out), n, c, out)
    out_pad = _make_sc_kernel(npad, npw, k, out)(tp, tq, ei_g, ej_g)
    return out_pad[:n].reshape(bsz, n, out)


# speedup vs baseline: 9.1426x; 1.0289x over previous
import functools

import jax
import jax.numpy as jnp
from jax import lax
from jax.experimental import pallas as pl
from jax.experimental.pallas import tpu as pltpu
from jax.experimental.pallas import tpu_sc as plsc

_NUM_CORES = 2
_NUM_SUBCORES = 16
_NW = _NUM_CORES * _NUM_SUBCORES
_L = 16
_G = 4


def _mm_body(x_ref, w_ref, b_ref, tp_ref, tq_ref):
    c = w_ref.shape[0] // 2
    w1 = w_ref[:c, :]
    w2 = w_ref[c:, :]
    xb = x_ref[...]
    tp_ref[...] = jnp.dot(xb, w1 - w2, preferred_element_type=jnp.float32) + b_ref[...]
    tq_ref[...] = jnp.dot(xb, w2, preferred_element_type=jnp.float32)


def _project(x2, W, b2, n, c, out):
    blk = 2000
    grid = (n // blk,)
    return pl.pallas_call(
        _mm_body,
        grid=grid,
        in_specs=[
            pl.BlockSpec((blk, c), lambda i: (i, 0)),
            pl.BlockSpec((2 * c, out), lambda i: (0, 0)),
            pl.BlockSpec((1, out), lambda i: (0, 0)),
        ],
        out_specs=[
            pl.BlockSpec((blk, out), lambda i: (i, 0)),
            pl.BlockSpec((blk, out), lambda i: (i, 0)),
        ],
        out_shape=[
            jax.ShapeDtypeStruct((n, out), jnp.float32),
            jax.ShapeDtypeStruct((n, out), jnp.float32),
        ],
    )(x2, W, b2)


def _make_sc_kernel(npad, npw, k, out):
    mesh = plsc.VectorSubcoreMesh(core_axis_name="c", subcore_axis_name="s")
    nch = out // _L
    gk = _G * k
    ngrp = npw // _G
    npair = ngrp // 2
    nq = npair // 8

    @functools.partial(
        pl.kernel,
        out_type=jax.ShapeDtypeStruct((npad, out), jnp.float32),
        mesh=mesh,
        scratch_types=[
            pltpu.VMEM((nq, 8, 2 * gk), jnp.int32),
            pltpu.VMEM((nq, 8, 2 * gk), jnp.int32),
            pltpu.VMEM((2, gk, out), jnp.float32),
            pltpu.VMEM((2, gk, out), jnp.float32),
            pltpu.VMEM((2 * _G, out), jnp.float32),
            pltpu.SemaphoreType.DMA,
            pltpu.SemaphoreType.DMA,
            pltpu.SemaphoreType.DMA,
            pltpu.SemaphoreType.DMA,
        ],
    )
    def sc_kernel(tp_hbm, tq_hbm, ei_hbm, ej_hbm, out_hbm,
                  ei_v, ej_v, bp, bq, ob, semp0, semp1, semq0, semq1):
        wid = lax.axis_index("s") * _NUM_CORES + lax.axis_index("c")
        base = wid * npw
        pltpu.sync_copy(ei_hbm.at[wid], ei_v)
        pltpu.sync_copy(ej_hbm.at[wid], ej_v)

        semp = (semp0, semp1)
        semq = (semq0, semq1)
        zero = jnp.zeros((_L,), jnp.float32)

        def issue(qq, rr, r):
            sl = pl.ds(r * gk, gk)
            pltpu.async_copy(tp_hbm.at[ei_v.at[qq, rr, sl]], bp.at[r], semp[r])
            pltpu.async_copy(tq_hbm.at[ej_v.at[qq, rr, sl]], bq.at[r], semq[r])

        def drain(qq, rr, r):
            sl = pl.ds(r * gk, gk)
            pltpu.make_async_copy(tp_hbm.at[ei_v.at[qq, rr, sl]], bp.at[r], semp[r]).wait()
            pltpu.make_async_copy(tq_hbm.at[ej_v.at[qq, rr, sl]], bq.at[r], semq[r]).wait()

        issue(0, 0, 0)

        def body(gp, carry):
            qq = gp // 8
            rr = lax.rem(gp, 8)
            for r in range(2):
                if r == 0:
                    issue(qq, rr, 1)
                else:
                    @pl.when(gp + 1 < npair)
                    def _():
                        issue((gp + 1) // 8, lax.rem(gp + 1, 8), 0)
                drain(qq, rr, r)

                def node(t, carry2):
                    for c in range(nch):
                        sl = pl.ds(c * _L, _L)
                        row = t * k
                        vals = [bp[r, row + kk, sl] + bq[r, row + kk, sl]
                                for kk in range(k)]
                        while len(vals) > 1:
                            vals = [jnp.maximum(vals[i], vals[i + 1])
                                    for i in range(0, len(vals), 2)]
                        ob[r * _G + t, sl] = jnp.maximum(vals[0], zero)
                    return carry2

                lax.fori_loop(0, _G, node, 0)
            pltpu.sync_copy(ob, out_hbm.at[pl.ds(base + gp * 2 * _G, 2 * _G)])
            return carry

        lax.fori_loop(0, npair, body, 0)

    return sc_kernel


def kernel(x, edge_index, W, b):
    bsz, n, c = x.shape
    k = edge_index.shape[-1]
    out = W.shape[1]

    x2 = x.reshape(n, c)
    ei = edge_index[1].reshape(n, k)
    ej = edge_index[0].reshape(n, k)

    npw = -(-n // (64 * _NW)) * 64
    npad = npw * _NW
    if npad != n:
        pad = ((0, npad - n), (0, 0))
        ei = jnp.pad(ei, pad)
        ej = jnp.pad(ej, pad)

    nq = npw * k // (8 * 2 * _G * k)
    ei_g = ei.reshape(_NW, nq, 8, 2 * _G * k)
    ej_g = ej.reshape(_NW, nq, 8, 2 * _G * k)

    tp, tq = _project(x2, W, b.reshape(1, out), n, c, out)
    out_pad = _make_sc_kernel(npad, npw, k, out)(tp, tq, ei_g, ej_g)
    return out_pad[:n].reshape(bsz, n, out)
